# baseline (speedup 1.0000x reference)
import functools

import jax
import jax.numpy as jnp
from jax import lax
from jax.experimental import pallas as pl
from jax.experimental.pallas import tpu as pltpu
from jax.experimental.pallas import tpu_sc as plsc

_N = 1000000
_K = 16
_B = 16384

try:
    _info = plsc.get_sparse_core_info()
    _NC, _NS = _info.num_cores, _info.num_subcores
except Exception:
    _NC, _NS = 2, 16
_NW = _NC * _NS
_BPW = _B // _NW

_mesh = plsc.VectorSubcoreMesh(core_axis_name="c", subcore_axis_name="s")


@functools.partial(
    pl.kernel,
    mesh=_mesh,
    out_type=(
        jax.ShapeDtypeStruct((_B, _K), jnp.float32),
        jax.ShapeDtypeStruct((_B, 1), jnp.float32),
    ),
    scratch_types=[
        pltpu.VMEM((_BPW,), jnp.int32),
        pltpu.VMEM((_BPW, _K), jnp.float32),
        pltpu.VMEM((_BPW, 1), jnp.float32),
        pltpu.SemaphoreType.DMA,
        pltpu.SemaphoreType.DMA,
    ],
    compiler_params=pltpu.CompilerParams(use_tc_tiling_on_sc=False),
)
def _gather_kernel(idx_hbm, pos_hbm, het_hbm, out_pos, out_het,
                   idx_v, pos_v, het_v, sem_p, sem_h):
    wid = lax.axis_index("s") * _NC + lax.axis_index("c")
    base = wid * _BPW
    pltpu.sync_copy(idx_hbm.at[pl.ds(base, _BPW)], idx_v)
    cp_p = pltpu.async_copy(pos_hbm.at[idx_v], pos_v, sem_p)
    cp_h = pltpu.async_copy(het_hbm.at[idx_v], het_v, sem_h)
    cp_p.wait()
    cp_h.wait()
    pltpu.sync_copy(pos_v, out_pos.at[pl.ds(base, _BPW)])
    pltpu.sync_copy(het_v, out_het.at[pl.ds(base, _BPW)])


def kernel(indices, values_pos, values_het):
    return _gather_kernel(indices.astype(jnp.int32), values_pos, values_het)


# speedup vs baseline: 2.4282x; 2.4282x over previous
import functools

import jax
import jax.numpy as jnp
from jax import lax
from jax.experimental import pallas as pl
from jax.experimental.pallas import tpu as pltpu
from jax.experimental.pallas import tpu_sc as plsc

_N = 1000000
_K = 16
_B = 16384

try:
    _info = plsc.get_sparse_core_info()
    _NC, _NS = _info.num_cores, _info.num_subcores
except Exception:
    _NC, _NS = 2, 16
_NW = _NC * _NS
_BPW = _B // _NW
_CHUNK = 128
_NCHUNK = _BPW // _CHUNK

_mesh = plsc.VectorSubcoreMesh(core_axis_name="c", subcore_axis_name="s")


@functools.partial(
    pl.kernel,
    mesh=_mesh,
    out_type=(
        jax.ShapeDtypeStruct((_B, _K), jnp.float32),
        jax.ShapeDtypeStruct((_B, 1), jnp.float32),
    ),
    scratch_types=[
        pltpu.VMEM((_BPW,), jnp.int32),
        pltpu.VMEM((_CHUNK, _K), jnp.float32),
        pltpu.VMEM((_CHUNK, 1), jnp.float32),
        pltpu.SemaphoreType.DMA,
        pltpu.SemaphoreType.DMA,
    ],
)
def _gather_kernel(idx_hbm, pos_hbm, het_hbm, out_pos, out_het,
                   idx_v, pos_v, het_v, sem_p, sem_h):
    wid = lax.axis_index("s") * _NC + lax.axis_index("c")
    base = wid * _BPW
    pltpu.sync_copy(idx_hbm.at[pl.ds(base, _BPW)], idx_v)

    def chunk_body(k):
        def group_body(g):
            vec = idx_v[pl.ds(k * _CHUNK + g * 16, 16)]
            for l in range(16):
                idx = vec[l]
                j = g * 16 + l
                pltpu.async_copy(pos_hbm.at[pl.ds(idx, 1)],
                                 pos_v.at[pl.ds(j, 1)], sem_p)
                pltpu.async_copy(het_hbm.at[pl.ds(idx, 1)],
                                 het_v.at[pl.ds(j, 1)], sem_h)

        pl.loop(0, _CHUNK // 16)(group_body)
        pltpu.make_async_copy(pos_hbm.at[pl.ds(0, _CHUNK)], pos_v,
                              sem_p).wait()
        pltpu.make_async_copy(het_hbm.at[pl.ds(0, _CHUNK)], het_v,
                              sem_h).wait()
        pltpu.sync_copy(pos_v, out_pos.at[pl.ds(base + k * _CHUNK, _CHUNK)])
        pltpu.sync_copy(het_v, out_het.at[pl.ds(base + k * _CHUNK, _CHUNK)])

    pl.loop(0, _NCHUNK)(chunk_body)


def kernel(indices, values_pos, values_het):
    return _gather_kernel(indices.astype(jnp.int32), values_pos, values_het)
